# Initial kernel scaffold; baseline (speedup 1.0000x reference)
import jax
import jax.numpy as jnp
from jax.experimental import pallas as pl


def kernel(inputs, length, table):
    raise NotImplementedError("write your pallas kernel here")


# speedup vs baseline: 6.6337x; 6.6337x over previous
import jax
import jax.numpy as jnp
from jax import lax
from jax.experimental import pallas as pl
from jax.experimental.pallas import tpu as pltpu
from jax.experimental.pallas import tpu_sc as plsc

SEQ = 128
DIM = 1280
NC = 2
NS = 16
JCH = SEQ // NC
ICH = SEQ // NS


def _body(table_hbm, idx_hbm, out_hbm, idx_v, rows_v, sem):
    c = lax.axis_index("c")
    s = lax.axis_index("s")
    pltpu.sync_copy(idx_hbm.at[c], idx_v)
    pltpu.async_copy(table_hbm.at[idx_v], rows_v, sem).wait()
    i0 = s * ICH
    jb = c * JCH
    copies = [
        pltpu.make_async_copy(rows_v, out_hbm.at[i0 + b, pl.ds(jb, JCH)], sem)
        for b in range(ICH)
    ]
    for h in copies:
        h.start()
    for h in copies:
        h.wait()


def kernel(inputs, length, table):
    del inputs
    idx = jnp.arange(SEQ, dtype=jnp.int32) + jnp.asarray(length, jnp.int32)
    idx = jnp.clip(idx, 0, SEQ - 1).reshape(NC, JCH)
    f = pl.kernel(
        _body,
        mesh=plsc.VectorSubcoreMesh(core_axis_name="c", subcore_axis_name="s"),
        out_type=jax.ShapeDtypeStruct((SEQ, SEQ, DIM), jnp.float32),
        scratch_types=[
            pltpu.VMEM((JCH,), jnp.int32),
            pltpu.VMEM((JCH, DIM), jnp.float32),
            pltpu.SemaphoreType.DMA,
        ],
    )
    return f(table, idx)


# speedup vs baseline: 7.3218x; 1.1037x over previous
import jax
import jax.numpy as jnp
from jax import lax
from jax.experimental import pallas as pl
from jax.experimental.pallas import tpu as pltpu
from jax.experimental.pallas import tpu_sc as plsc

SEQ = 128
DIM = 1280
NC = 2
NS = 16
NW = NC * NS
RCH = SEQ // NW
PAD = 8
IBLK = 8


def _sc_gather_body(table_hbm, idx_hbm, rows_hbm, idx_v, rows_v, sem):
    w = lax.axis_index("s") * NC + lax.axis_index("c")
    pltpu.sync_copy(idx_hbm.at[w], idx_v)
    pltpu.async_copy(table_hbm.at[idx_v], rows_v, sem).wait()
    pltpu.sync_copy(rows_v.at[pl.ds(0, RCH)], rows_hbm.at[pl.ds(w * RCH, RCH)])


def _tc_broadcast_body(rows_ref, out_ref):
    out_ref[...] = jnp.broadcast_to(rows_ref[...], (IBLK, SEQ, DIM))


def kernel(inputs, length, table):
    del inputs
    idx = jnp.arange(SEQ, dtype=jnp.int32) + jnp.asarray(length, jnp.int32)
    idx = jnp.clip(idx, 0, SEQ - 1).reshape(NW, RCH)
    idx = jnp.concatenate([idx, idx], axis=1)

    gather = pl.kernel(
        _sc_gather_body,
        mesh=plsc.VectorSubcoreMesh(core_axis_name="c", subcore_axis_name="s"),
        out_type=jax.ShapeDtypeStruct((SEQ, DIM), jnp.float32),
        scratch_types=[
            pltpu.VMEM((PAD,), jnp.int32),
            pltpu.VMEM((PAD, DIM), jnp.float32),
            pltpu.SemaphoreType.DMA,
        ],
    )
    rows = gather(table, idx)

    return pl.pallas_call(
        _tc_broadcast_body,
        grid=(SEQ // IBLK,),
        in_specs=[pl.BlockSpec((SEQ, DIM), lambda i: (0, 0))],
        out_specs=pl.BlockSpec((IBLK, SEQ, DIM), lambda i: (i, 0, 0)),
        out_shape=jax.ShapeDtypeStruct((SEQ, SEQ, DIM), jnp.float32),
    )(rows)
